# baseline (speedup 1.0000x reference)
import jax
import jax.numpy as jnp
from jax import lax
from jax.experimental import pallas as pl
from jax.experimental.pallas import tpu as pltpu
from jax.experimental.pallas import tpu_sc as plsc

N0 = 10000
N1 = 10000
E = 320000
D = 128
DE = 16
H = 128

NC = 2
NS = 16
NW = NC * NS
CHUNK = 128
KP = 80
G = 8
EPW = KP * CHUNK
E_PAD = NW * EPW
N1_PAD = 10240
DEW = 128
RPS = N1_PAD // NS


def _emb_body(x0_ref, w0_ref, b0_ref, x1_ref, w1_ref, b1_ref, h0_ref, h1_ref):
    h0_ref[...] = (
        jnp.dot(x0_ref[...], w0_ref[...], preferred_element_type=jnp.float32)
        + b0_ref[...]
    )
    h1_ref[...] = (
        jnp.dot(x1_ref[...], w1_ref[...], preferred_element_type=jnp.float32)
        + b1_ref[...]
    )


def _out_body(h1_ref, s_ref, t_ref, we_ref, wagg_ref, bagg_ref, o_ref):
    s = s_ref[0, :N1, :] + s_ref[1, :N1, :]
    t = t_ref[0, :N1, :DE] + t_ref[1, :N1, :DE]
    w2 = jnp.dot(we_ref[...], wagg_ref[...], preferred_element_type=jnp.float32)
    acc = jnp.dot(s, wagg_ref[...], preferred_element_type=jnp.float32) + jnp.dot(
        t, w2, preferred_element_type=jnp.float32
    )
    o_ref[...] = jnp.maximum(h1_ref[...] + acc + bagg_ref[...], 0.0)


def _pad_body(x_ref, o_ref):
    o_ref[...] = jnp.concatenate(
        [x_ref[...], jnp.zeros((x_ref.shape[0], DEW - DE), jnp.float32)], axis=1
    )


def _sc_s_body(
    h0_hbm,
    src_hbm,
    dst_hbm,
    iota_hbm,
    s_out,
    acc_s,
    src_v,
    dst_v,
    rows_v0,
    rows_v1,
    sem0,
    sem1,
):
    c = lax.axis_index("c")
    s = lax.axis_index("s")
    wid = s * NC + c

    zf = jnp.zeros((16,), jnp.float32)

    @pl.loop(0, CHUNK)
    def _(r):
        @pl.loop(0, H, step=16)
        def _(col):
            rows_v0[r, pl.ds(col, 16)] = zf

    kps = RPS // CHUNK
    pltpu.sync_copy(iota_hbm.at[s], dst_v)

    @pl.loop(0, kps)
    def _(k):
        pltpu.sync_copy(rows_v0, acc_s.at[dst_v.at[k]])

    plsc.subcore_barrier()

    bufs = (rows_v0, rows_v1)
    sems = (sem0, sem1)

    @pl.loop(0, KP, step=G)
    def _(j0):
        pltpu.sync_copy(src_hbm.at[wid, pl.ds(j0, G)], src_v)
        pltpu.sync_copy(dst_hbm.at[wid, pl.ds(j0, G)], dst_v)
        descs = {0: pltpu.async_copy(h0_hbm.at[src_v.at[0]], bufs[0], sems[0])}
        for jj in range(G):
            descs[jj].wait()
            if jj + 1 < G:
                descs[jj + 1] = pltpu.async_copy(
                    h0_hbm.at[src_v.at[jj + 1]], bufs[(jj + 1) % 2], sems[(jj + 1) % 2]
                )
            pltpu.sync_copy(bufs[jj % 2], acc_s.at[dst_v.at[jj]], add=True)

    plsc.subcore_barrier()

    pltpu.sync_copy(iota_hbm.at[s], dst_v)

    @pl.loop(0, kps)
    def _(k):
        pltpu.sync_copy(acc_s.at[dst_v.at[k]], rows_v0)
        pltpu.sync_copy(rows_v0, s_out.at[c, pl.ds(s * RPS + k * CHUNK, CHUNK)])


def _sc_t_body(
    dst_hbm,
    ea_hbm,
    iota_hbm,
    t_out,
    acc_t,
    dst_v,
    ea_v0,
    ea_v1,
    sem0,
    sem1,
):
    c = lax.axis_index("c")
    s = lax.axis_index("s")
    wid = s * NC + c

    zf = jnp.zeros((16,), jnp.float32)

    @pl.loop(0, CHUNK)
    def _(r):
        @pl.loop(0, DEW, step=16)
        def _(col):
            ea_v0[r, pl.ds(col, 16)] = zf

    kps = RPS // CHUNK
    pltpu.sync_copy(iota_hbm.at[s], dst_v)

    @pl.loop(0, kps)
    def _(k):
        pltpu.sync_copy(ea_v0, acc_t.at[dst_v.at[k]])

    plsc.subcore_barrier()

    bufs = (ea_v0, ea_v1)
    sems = (sem0, sem1)

    @pl.loop(0, KP, step=G)
    def _(j0):
        pltpu.sync_copy(dst_hbm.at[wid, pl.ds(j0, G)], dst_v)
        descs = {0: pltpu.async_copy(ea_hbm.at[wid, j0], bufs[0], sems[0])}
        for jj in range(G):
            descs[jj].wait()
            if jj + 1 < G:
                descs[jj + 1] = pltpu.async_copy(
                    ea_hbm.at[wid, j0 + jj + 1], bufs[(jj + 1) % 2], sems[(jj + 1) % 2]
                )
            pltpu.sync_copy(bufs[jj % 2], acc_t.at[dst_v.at[jj]], add=True)

    plsc.subcore_barrier()

    pltpu.sync_copy(iota_hbm.at[s], dst_v)

    @pl.loop(0, kps)
    def _(k):
        pltpu.sync_copy(acc_t.at[dst_v.at[k]], ea_v0)
        pltpu.sync_copy(ea_v0, t_out.at[c, pl.ds(s * RPS + k * CHUNK, CHUNK)])


def _sc_segment_sums(h0, src_p, dst_p, ea_p, iota3d):
    mesh = plsc.VectorSubcoreMesh(
        core_axis_name="c", subcore_axis_name="s", num_cores=NC, num_subcores=NS
    )
    run_s = pl.kernel(
        _sc_s_body,
        out_type=jax.ShapeDtypeStruct((NC, N1_PAD, H), jnp.float32),
        mesh=mesh,
        scratch_types=[
            pltpu.VMEM_SHARED((N1_PAD, H), jnp.float32),
            pltpu.VMEM((G, CHUNK), jnp.int32),
            pltpu.VMEM((G, CHUNK), jnp.int32),
            pltpu.VMEM((CHUNK, H), jnp.float32),
            pltpu.VMEM((CHUNK, H), jnp.float32),
            pltpu.SemaphoreType.DMA,
            pltpu.SemaphoreType.DMA,
        ],
    )
    run_t = pl.kernel(
        _sc_t_body,
        out_type=jax.ShapeDtypeStruct((NC, N1_PAD, DEW), jnp.float32),
        mesh=mesh,
        scratch_types=[
            pltpu.VMEM_SHARED((N1_PAD, DEW), jnp.float32),
            pltpu.VMEM((G, CHUNK), jnp.int32),
            pltpu.VMEM((CHUNK, DEW), jnp.float32),
            pltpu.VMEM((CHUNK, DEW), jnp.float32),
            pltpu.SemaphoreType.DMA,
            pltpu.SemaphoreType.DMA,
        ],
    )
    return run_s(h0, src_p, dst_p, iota3d), run_t(dst_p, ea_p, iota3d)


def kernel(x0, x1, edge_index, edge_attr, W0, b0, W1, b1, W_e, W_agg, b_agg):
    x0 = x0.astype(jnp.float32)
    x1 = x1.astype(jnp.float32)
    edge_attr = edge_attr.astype(jnp.float32)

    h0, h1 = pl.pallas_call(
        _emb_body,
        out_shape=(
            jax.ShapeDtypeStruct((N0, H), jnp.float32),
            jax.ShapeDtypeStruct((N1, H), jnp.float32),
        ),
    )(x0, W0, b0.reshape(1, H), x1, W1, b1.reshape(1, H))

    pad = E_PAD - E
    src = edge_index[0].astype(jnp.int32)
    dst = edge_index[1].astype(jnp.int32)
    src_p = jnp.concatenate([src, jnp.zeros((pad,), jnp.int32)]).reshape(
        NW, KP, CHUNK
    )
    dst_p = jnp.concatenate([dst, jnp.full((pad,), N1, jnp.int32)]).reshape(
        NW, KP, CHUNK
    )
    ea_r = jnp.concatenate([edge_attr, jnp.zeros((pad, DE), jnp.float32)])
    blk = E_PAD // 40
    ea_w = pl.pallas_call(
        _pad_body,
        grid=(40,),
        in_specs=[pl.BlockSpec((blk, DE), lambda i: (i, 0))],
        out_specs=pl.BlockSpec((blk, DEW), lambda i: (i, 0)),
        out_shape=jax.ShapeDtypeStruct((E_PAD, DEW), jnp.float32),
    )(ea_r)
    ea_p = ea_w.reshape(NW, KP, CHUNK, DEW)

    iota2d = jnp.arange(N1_PAD, dtype=jnp.int32).reshape(
        NS, N1_PAD // NS // CHUNK, CHUNK
    )
    iota2d = jnp.pad(iota2d, ((0, 0), (0, G - N1_PAD // NS // CHUNK), (0, 0)),
                     mode="edge")
    s_part, t_part = _sc_segment_sums(h0, src_p, dst_p, ea_p, iota2d)

    out = pl.pallas_call(
        _out_body,
        out_shape=jax.ShapeDtypeStruct((N1, H), jnp.float32),
    )(h1, s_part, t_part, W_e, W_agg, b_agg.reshape(1, H))
    return out


# speedup vs baseline: 1.1045x; 1.1045x over previous
import jax
import jax.numpy as jnp
from jax import lax
from jax.experimental import pallas as pl
from jax.experimental.pallas import tpu as pltpu
from jax.experimental.pallas import tpu_sc as plsc

N0 = 10000
N1 = 10000
E = 320000
D = 128
DE = 16
H = 128

NC = 2
NS = 16
NW = NC * NS
CHUNK = 128
KP = 80
G = 8
EPW = KP * CHUNK
E_PAD = NW * EPW
N1_PAD = 10240
DEW = 128
RPS = N1_PAD // NS


def _emb_body(x0_ref, w0_ref, b0_ref, x1_ref, w1_ref, b1_ref, h0_ref, h1_ref):
    h0_ref[...] = (
        jnp.dot(x0_ref[...], w0_ref[...], preferred_element_type=jnp.float32)
        + b0_ref[...]
    )
    h1_ref[...] = (
        jnp.dot(x1_ref[...], w1_ref[...], preferred_element_type=jnp.float32)
        + b1_ref[...]
    )


def _out_body(h1_ref, s_ref, t_ref, we_ref, wagg_ref, bagg_ref, o_ref):
    s = s_ref[0, :N1, :] + s_ref[1, :N1, :]
    t = t_ref[0, :N1, :DE] + t_ref[1, :N1, :DE]
    w2 = jnp.dot(we_ref[...], wagg_ref[...], preferred_element_type=jnp.float32)
    acc = jnp.dot(s, wagg_ref[...], preferred_element_type=jnp.float32) + jnp.dot(
        t, w2, preferred_element_type=jnp.float32
    )
    o_ref[...] = jnp.maximum(h1_ref[...] + acc + bagg_ref[...], 0.0)


def _sc_s_body(
    h0_hbm,
    src_hbm,
    dst_hbm,
    iota_hbm,
    s_out,
    acc_s,
    src_v,
    dst_v,
    rows_v0,
    rows_v1,
    sem0,
    sem1,
):
    c = lax.axis_index("c")
    s = lax.axis_index("s")
    wid = s * NC + c

    zf = jnp.zeros((16,), jnp.float32)

    @pl.loop(0, CHUNK)
    def _(r):
        @pl.loop(0, H, step=16)
        def _(col):
            rows_v0[r, pl.ds(col, 16)] = zf

    kps = RPS // CHUNK
    pltpu.sync_copy(iota_hbm.at[s], dst_v)

    @pl.loop(0, kps)
    def _(k):
        pltpu.sync_copy(rows_v0, acc_s.at[dst_v.at[k]])

    plsc.subcore_barrier()

    bufs = (rows_v0, rows_v1)
    sems = (sem0, sem1)

    @pl.loop(0, KP, step=G)
    def _(j0):
        pltpu.sync_copy(src_hbm.at[wid, pl.ds(j0, G)], src_v)
        pltpu.sync_copy(dst_hbm.at[wid, pl.ds(j0, G)], dst_v)
        descs = {0: pltpu.async_copy(h0_hbm.at[src_v.at[0]], bufs[0], sems[0])}
        for jj in range(G):
            descs[jj].wait()
            if jj + 1 < G:
                descs[jj + 1] = pltpu.async_copy(
                    h0_hbm.at[src_v.at[jj + 1]], bufs[(jj + 1) % 2], sems[(jj + 1) % 2]
                )
            pltpu.sync_copy(bufs[jj % 2], acc_s.at[dst_v.at[jj]], add=True)

    plsc.subcore_barrier()

    pltpu.sync_copy(iota_hbm.at[s], dst_v)

    @pl.loop(0, kps)
    def _(k):
        pltpu.sync_copy(acc_s.at[dst_v.at[k]], rows_v0)
        pltpu.sync_copy(rows_v0, s_out.at[c, pl.ds(s * RPS + k * CHUNK, CHUNK)])


def _sc_t_body(
    dst_hbm,
    ea_hbm,
    iota_hbm,
    t_out,
    acc_t,
    dst_v,
    ea_v0,
    ea_v1,
    sem0,
    sem1,
):
    c = lax.axis_index("c")
    s = lax.axis_index("s")
    wid = s * NC + c

    zf = jnp.zeros((16,), jnp.float32)

    @pl.loop(0, CHUNK)
    def _(r):
        @pl.loop(0, DEW, step=16)
        def _(col):
            ea_v0[r, pl.ds(col, 16)] = zf

    kps = RPS // CHUNK
    pltpu.sync_copy(iota_hbm.at[s], dst_v)

    @pl.loop(0, kps)
    def _(k):
        pltpu.sync_copy(ea_v0, acc_t.at[dst_v.at[k]])

    plsc.subcore_barrier()

    bufs = (ea_v0, ea_v1)
    sems = (sem0, sem1)

    @pl.loop(0, KP, step=G)
    def _(j0):
        pltpu.sync_copy(dst_hbm.at[wid, pl.ds(j0, G)], dst_v)
        descs = {0: pltpu.async_copy(ea_hbm.at[wid, j0], bufs[0], sems[0])}
        for jj in range(G):
            descs[jj].wait()
            if jj + 1 < G:
                descs[jj + 1] = pltpu.async_copy(
                    ea_hbm.at[wid, j0 + jj + 1], bufs[(jj + 1) % 2], sems[(jj + 1) % 2]
                )
            pltpu.sync_copy(bufs[jj % 2], acc_t.at[dst_v.at[jj]], add=True)

    plsc.subcore_barrier()

    pltpu.sync_copy(iota_hbm.at[s], dst_v)

    @pl.loop(0, kps)
    def _(k):
        pltpu.sync_copy(acc_t.at[dst_v.at[k]], ea_v0)
        pltpu.sync_copy(ea_v0, t_out.at[c, pl.ds(s * RPS + k * CHUNK, CHUNK)])


def _sc_segment_sums(h0, src_p, dst_p, ea_p, iota3d):
    mesh = plsc.VectorSubcoreMesh(
        core_axis_name="c", subcore_axis_name="s", num_cores=NC, num_subcores=NS
    )
    run_s = pl.kernel(
        _sc_s_body,
        out_type=jax.ShapeDtypeStruct((NC, N1_PAD, H), jnp.float32),
        mesh=mesh,
        scratch_types=[
            pltpu.VMEM_SHARED((N1_PAD, H), jnp.float32),
            pltpu.VMEM((G, CHUNK), jnp.int32),
            pltpu.VMEM((G, CHUNK), jnp.int32),
            pltpu.VMEM((CHUNK, H), jnp.float32),
            pltpu.VMEM((CHUNK, H), jnp.float32),
            pltpu.SemaphoreType.DMA,
            pltpu.SemaphoreType.DMA,
        ],
    )
    run_t = pl.kernel(
        _sc_t_body,
        out_type=jax.ShapeDtypeStruct((NC, N1_PAD, DEW), jnp.float32),
        mesh=mesh,
        scratch_types=[
            pltpu.VMEM_SHARED((N1_PAD, DEW), jnp.float32),
            pltpu.VMEM((G, CHUNK), jnp.int32),
            pltpu.VMEM((CHUNK, DEW), jnp.float32),
            pltpu.VMEM((CHUNK, DEW), jnp.float32),
            pltpu.SemaphoreType.DMA,
            pltpu.SemaphoreType.DMA,
        ],
    )
    return run_s(h0, src_p, dst_p, iota3d), run_t(dst_p, ea_p, iota3d)


def kernel(x0, x1, edge_index, edge_attr, W0, b0, W1, b1, W_e, W_agg, b_agg):
    x0 = x0.astype(jnp.float32)
    x1 = x1.astype(jnp.float32)
    edge_attr = edge_attr.astype(jnp.float32)

    h0, h1 = pl.pallas_call(
        _emb_body,
        out_shape=(
            jax.ShapeDtypeStruct((N0, H), jnp.float32),
            jax.ShapeDtypeStruct((N1, H), jnp.float32),
        ),
    )(x0, W0, b0.reshape(1, H), x1, W1, b1.reshape(1, H))

    pad = E_PAD - E
    src = edge_index[0].astype(jnp.int32)
    dst = edge_index[1].astype(jnp.int32)
    src_p = jnp.concatenate([src, jnp.zeros((pad,), jnp.int32)]).reshape(
        NW, KP, CHUNK
    )
    dst_p = jnp.concatenate([dst, jnp.full((pad,), N1, jnp.int32)]).reshape(
        NW, KP, CHUNK
    )
    ea_w = jnp.pad(edge_attr, ((0, pad), (0, DEW - DE)))
    ea_p = ea_w.reshape(NW, KP, CHUNK, DEW)

    iota2d = jnp.arange(N1_PAD, dtype=jnp.int32).reshape(
        NS, N1_PAD // NS // CHUNK, CHUNK
    )
    iota2d = jnp.pad(iota2d, ((0, 0), (0, G - N1_PAD // NS // CHUNK), (0, 0)),
                     mode="edge")
    s_part, t_part = _sc_segment_sums(h0, src_p, dst_p, ea_p, iota2d)

    out = pl.pallas_call(
        _out_body,
        out_shape=jax.ShapeDtypeStruct((N1, H), jnp.float32),
    )(h1, s_part, t_part, W_e, W_agg, b_agg.reshape(1, H))
    return out
